# Initial kernel scaffold; baseline (speedup 1.0000x reference)
import jax
import jax.numpy as jnp
from jax.experimental import pallas as pl


def kernel(x, j_idx, k_idx, W1, b1, W2, b2):
    raise NotImplementedError("write your pallas kernel here")


# speedup vs baseline: 23.4738x; 23.4738x over previous
import functools

import jax
import jax.numpy as jnp
from jax import lax
from jax.experimental import pallas as pl
from jax.experimental.pallas import tpu as pltpu
from jax.experimental.pallas import tpu_sc as plsc

B, N, D, S, DOUT = 2, 65536, 16, 8, 16
NS = N * S
DH = 2 * D
LW = 128

NBP = 4096


def _tc_project_body(x_ref, w1_ref, b1_ref, tp_ref):
    xa = x_ref[...]
    w1 = w1_ref[...]
    pj = jnp.dot(xa, w1[D:2 * D], preferred_element_type=jnp.float32)
    pk = jnp.dot(xa, w1[2 * D:], preferred_element_type=jnp.float32)
    pi = jnp.dot(xa, w1[:D], preferred_element_type=jnp.float32) + b1_ref[...]
    z = jnp.zeros((NBP, DH), jnp.float32)
    tp_ref[...] = jnp.concatenate([pj, pk, pi, z], axis=1)


_tc_project = pl.pallas_call(
    _tc_project_body,
    grid=((B * N) // NBP,),
    in_specs=[
        pl.BlockSpec((NBP, D), lambda i: (i, 0)),
        pl.BlockSpec((3 * D, DH), lambda i: (0, 0)),
        pl.BlockSpec((1, DH), lambda i: (0, 0)),
    ],
    out_specs=pl.BlockSpec((NBP, LW), lambda i: (i, 0)),
    out_shape=jax.ShapeDtypeStruct((B * N, LW), jnp.float32),
)

NC, NSUB = 2, 16
NW = NC * NSUB
PW = NS // NW
GI = 128
QO = 4
CO = GI * QO
CHUNKS = PW // CO
NCH = NS // CO
GROWS = NS // QO


def _sc_gather_body(tp_hbm, jk_hbm, g_hbm, idx_v, pk_v, pb_v, sem):
    cid = lax.axis_index("c")
    sid = lax.axis_index("s")
    wid = sid * NC + cid

    for t in range(2):
        lo = t * DH
        for b in range(B):
            tb = t * B + b

            def chunk_body(g, _, tb=tb, b=b, lo=lo):
                ch = wid * CHUNKS + g
                r0 = (tb * NCH + ch) * QO
                pltpu.sync_copy(jk_hbm.at[pl.ds(r0, QO)], idx_v)
                cps = [
                    pltpu.async_copy(
                        tp_hbm.at[idx_v.at[q]],
                        pk_v.at[pl.ds(q * GI, GI)],
                        sem,
                    )
                    for q in range(QO)
                ]
                for cp in cps:
                    cp.wait()
                for q in range(QO):
                    for r in range(GI):
                        for h in range(2):
                            pb_v[r, pl.ds(q * DH + h * 16, 16)] = (
                                pk_v[q * GI + r, pl.ds(lo + h * 16, 16)]
                            )
                o0 = tb * GROWS + ch * GI
                pltpu.sync_copy(pb_v, g_hbm.at[pl.ds(o0, GI)])
                return 0

            lax.fori_loop(0, CHUNKS, chunk_body, 0, unroll=False)


@functools.cache
def _sc_gather():
    return pl.kernel(
        _sc_gather_body,
        out_type=jax.ShapeDtypeStruct((2 * B * GROWS, LW), jnp.float32),
        mesh=plsc.VectorSubcoreMesh(core_axis_name="c", subcore_axis_name="s"),
        scratch_types=[
            pltpu.VMEM((QO, GI), jnp.int32),
            pltpu.VMEM((CO, LW), jnp.float32),
            pltpu.VMEM((GI, LW), jnp.float32),
            pltpu.SemaphoreType.DMA,
        ],
    )


NB = 2048
GRID = (B * N) // NB
RPN = S // QO
_INV_SQRT2 = 0.7071067811865476


def _tc_combine_body(tp_ref, gj_ref, gk_ref, w2r_ref, b2_ref, out_ref):
    a = tp_ref[...][:, 2 * DH:3 * DH]
    a4 = jnp.concatenate([a] * QO, axis=1)
    a8 = jnp.repeat(a4, RPN, axis=0)
    t = gj_ref[...] + gk_ref[...] + a8
    h = 0.5 * t * (1.0 + lax.erf(t * _INV_SQRT2))
    o = jnp.dot(h, w2r_ref[...], preferred_element_type=jnp.float32)
    o = o.reshape(NB, RPN, DOUT).sum(axis=1)
    out_ref[...] = o + b2_ref[...]


_tc_combine = pl.pallas_call(
    _tc_combine_body,
    grid=(GRID,),
    in_specs=[
        pl.BlockSpec((NB, LW), lambda i: (i, 0)),
        pl.BlockSpec((RPN * NB, LW), lambda i: (i, 0)),
        pl.BlockSpec((RPN * NB, LW), lambda i: (i, 0)),
        pl.BlockSpec((LW, DOUT), lambda i: (0, 0)),
        pl.BlockSpec((1, DOUT), lambda i: (0, 0)),
    ],
    out_specs=pl.BlockSpec((NB, DOUT), lambda i: (i, 0)),
    out_shape=jax.ShapeDtypeStruct((B * N, DOUT), jnp.float32),
)


def kernel(x, j_idx, k_idx, W1, b1, W2, b2):
    off = (jnp.arange(B, dtype=jnp.int32) * N)[None, :, None]
    jk = (
        (jnp.stack([j_idx.reshape(B, NS), k_idx.reshape(B, NS)]) + off)
        .reshape(2, B, NCH, GI, QO)
        .swapaxes(3, 4)
        .reshape(2 * B * NCH * QO, GI)
    )
    tp = _tc_project(x.reshape(B * N, D), W1, b1.reshape(1, DH))
    g = _sc_gather()(tp, jk)
    gj = g[: B * GROWS]
    gk = g[B * GROWS:]
    w2r = jnp.concatenate([W2] * QO, axis=0) / S
    out = _tc_combine(tp, gj, gk, w2r, b2.reshape(1, DOUT))
    return out.reshape(B, N, DOUT)
